# baseline (speedup 1.0000x reference)
import functools

import jax
import jax.numpy as jnp
from jax import lax
from jax.experimental import pallas as pl
from jax.experimental.pallas import tpu as pltpu

N = 10000
NE = 320000
D = 128
V = 42
H = 64
EBLK = 8000
NBLK = 2000
EPS = 1e-5


def _swish(x):
    return x * jax.nn.sigmoid(x)


def _pad8(b):
    return jnp.zeros((8, H), jnp.float32).at[0].set(b)


def _proj_body(x_ref, wa_ref, wb_ref, a_ref, b_ref):
    x = x_ref[...]
    a_ref[...] = jnp.dot(x, wa_ref[...], preferred_element_type=jnp.float32)
    b_ref[...] = jnp.dot(x, wb_ref[...], preferred_element_type=jnp.float32)


def _proj(x2, w0a, w0b):
    g = N // NBLK
    return pl.pallas_call(
        _proj_body,
        grid=(g,),
        in_specs=[
            pl.BlockSpec((NBLK, D), lambda i: (i, 0)),
            pl.BlockSpec((D, H), lambda i: (0, 0)),
            pl.BlockSpec((D, H), lambda i: (0, 0)),
        ],
        out_specs=[
            pl.BlockSpec((NBLK, H), lambda i: (i, 0)),
            pl.BlockSpec((NBLK, H), lambda i: (i, 0)),
        ],
        out_shape=[
            jax.ShapeDtypeStruct((N, H), jnp.float32),
            jax.ShapeDtypeStruct((N, H), jnp.float32),
        ],
    )(x2, w0a, w0b)


def _e1_body(ga_ref, gb_ref, b_ref, h_ref, st_ref, acc):
    i = pl.program_id(0)
    h = _swish(ga_ref[...] + gb_ref[...] + b_ref[0:1, :])
    h_ref[...] = h

    @pl.when(i == 0)
    def _():
        acc[...] = jnp.zeros_like(acc)

    acc[0:1, :] = acc[0:1, :] + jnp.sum(h, axis=0, keepdims=True)
    acc[1:2, :] = acc[1:2, :] + jnp.sum(h * h, axis=0, keepdims=True)

    @pl.when(i == pl.num_programs(0) - 1)
    def _():
        st_ref[...] = acc[...]


def _e1(ga, gb, b0p):
    g = NE // EBLK
    return pl.pallas_call(
        _e1_body,
        grid=(g,),
        in_specs=[
            pl.BlockSpec((EBLK, H), lambda i: (i, 0)),
            pl.BlockSpec((EBLK, H), lambda i: (i, 0)),
            pl.BlockSpec((8, H), lambda i: (0, 0)),
        ],
        out_specs=[
            pl.BlockSpec((EBLK, H), lambda i: (i, 0)),
            pl.BlockSpec((8, H), lambda i: (0, 0)),
        ],
        out_shape=[
            jax.ShapeDtypeStruct((NE, H), jnp.float32),
            jax.ShapeDtypeStruct((8, H), jnp.float32),
        ],
        scratch_shapes=[pltpu.VMEM((8, H), jnp.float32)],
    )(ga, gb, b0p)


def _em_body(x_ref, w_ref, b_ref, y_ref, st_ref, acc):
    i = pl.program_id(0)
    y = _swish(
        jnp.dot(x_ref[...], w_ref[...], preferred_element_type=jnp.float32)
        + b_ref[0:1, :]
    )
    y_ref[...] = y

    @pl.when(i == 0)
    def _():
        acc[...] = jnp.zeros_like(acc)

    acc[0:1, :] = acc[0:1, :] + jnp.sum(y, axis=0, keepdims=True)
    acc[1:2, :] = acc[1:2, :] + jnp.sum(y * y, axis=0, keepdims=True)

    @pl.when(i == pl.num_programs(0) - 1)
    def _():
        st_ref[...] = acc[...]


def _em_last_body(x_ref, w_ref, b_ref, y_ref):
    y_ref[...] = _swish(
        jnp.dot(x_ref[...], w_ref[...], preferred_element_type=jnp.float32)
        + b_ref[0:1, :]
    )


def _em(x, w, bp, stats):
    g = NE // EBLK
    in_specs = [
        pl.BlockSpec((EBLK, H), lambda i: (i, 0)),
        pl.BlockSpec((H, H), lambda i: (0, 0)),
        pl.BlockSpec((8, H), lambda i: (0, 0)),
    ]
    if stats:
        return pl.pallas_call(
            _em_body,
            grid=(g,),
            in_specs=in_specs,
            out_specs=[
                pl.BlockSpec((EBLK, H), lambda i: (i, 0)),
                pl.BlockSpec((8, H), lambda i: (0, 0)),
            ],
            out_shape=[
                jax.ShapeDtypeStruct((NE, H), jnp.float32),
                jax.ShapeDtypeStruct((8, H), jnp.float32),
            ],
            scratch_shapes=[pltpu.VMEM((8, H), jnp.float32)],
        )(x, w, bp)
    return pl.pallas_call(
        _em_last_body,
        grid=(g,),
        in_specs=in_specs,
        out_specs=pl.BlockSpec((EBLK, H), lambda i: (i, 0)),
        out_shape=jax.ShapeDtypeStruct((NE, H), jnp.float32),
    )(x, w, bp)


def _fold(st, g, be, w_next, b_next):
    mu = st[0] / NE
    var = st[1] / NE - mu * mu
    s = g / jnp.sqrt(var + EPS)
    wf = s[:, None] * w_next
    bf = b_next + (be - mu * s) @ w_next
    return wf, bf


def _upd_body(x1_ref, x2_ref, s2_ref, c2_ref,
              wa_ref, wb_ref, wc_ref, b0_ref, w1_ref, b1_ref,
              w2_ref, b2_ref, w3_ref, b3_ref, xo_ref):
    cnt = c2_ref[0, :, 0:1] + c2_ref[1, :, 0:1]
    cnt = jnp.maximum(cnt, 1.0)
    agg = (s2_ref[0] + s2_ref[1]) / cnt
    t = _swish(
        jnp.dot(x1_ref[...], wa_ref[...], preferred_element_type=jnp.float32)
        + jnp.dot(x2_ref[...], wb_ref[...], preferred_element_type=jnp.float32)
        + jnp.dot(agg, wc_ref[...], preferred_element_type=jnp.float32)
        + b0_ref[0:1, :]
    )
    t = _swish(jnp.dot(t, w1_ref[...], preferred_element_type=jnp.float32) + b1_ref[0:1, :])
    t = _swish(jnp.dot(t, w2_ref[...], preferred_element_type=jnp.float32) + b2_ref[0:1, :])
    t = _swish(jnp.dot(t, w3_ref[...], preferred_element_type=jnp.float32) + b3_ref[0:1, :])
    xo_ref[...] = x2_ref[...] + t


def _upd(x1p, x2, s2, c2, upd_p):
    wu0 = upd_p['W'][0]
    wa = jnp.zeros((H, H), jnp.float32).at[:V].set(wu0[:V])
    wb = wu0[V:V + D]
    wc = wu0[V + D:]
    g = N // NBLK
    cst = lambda shape: pl.BlockSpec(shape, lambda i: tuple(0 for _ in shape))
    return pl.pallas_call(
        _upd_body,
        grid=(g,),
        in_specs=[
            pl.BlockSpec((NBLK, H), lambda i: (i, 0)),
            pl.BlockSpec((NBLK, D), lambda i: (i, 0)),
            pl.BlockSpec((2, NBLK, H), lambda i: (0, i, 0)),
            pl.BlockSpec((2, NBLK, 16), lambda i: (0, i, 0)),
            cst((H, H)), cst((D, H)), cst((H, H)), cst((8, H)),
            cst((H, H)), cst((8, H)),
            cst((H, H)), cst((8, H)),
            cst((H, D)), cst((8, D)),
        ],
        out_specs=pl.BlockSpec((NBLK, D), lambda i: (i, 0)),
        out_shape=jax.ShapeDtypeStruct((N, D), jnp.float32),
    )(x1p, x2, s2, c2,
      wa, wb, wc, _pad8(upd_p['b'][0]),
      upd_p['W'][1], _pad8(upd_p['b'][1]),
      upd_p['W'][2], _pad8(upd_p['b'][2]),
      upd_p['W'][3],
      jnp.zeros((8, D), jnp.float32).at[0].set(upd_p['b'][3]))


def _final_body(x2_ref, x1h_ref,
                wp0_ref, bp0_ref, wp1_ref, bp1_ref, wp2_ref, bp2_ref,
                wp3_ref, bp3_ref, wq0_ref, bq0_ref, wq1_ref, bq1_ref,
                out_ref, acc):
    i = pl.program_id(0)
    t = _swish(jnp.dot(x2_ref[...], wp0_ref[...], preferred_element_type=jnp.float32) + bp0_ref[0:1, :])
    t = _swish(jnp.dot(t, wp1_ref[...], preferred_element_type=jnp.float32) + bp1_ref[0:1, :])
    t = _swish(jnp.dot(t, wp2_ref[...], preferred_element_type=jnp.float32) + bp2_ref[0:1, :])
    h = jnp.dot(t, wp3_ref[...], preferred_element_type=jnp.float32) + bp3_ref[0:1, :]

    @pl.when(i == 0)
    def _():
        acc[...] = jnp.zeros_like(acc)

    acc[0:1, :] = acc[0:1, :] + jnp.sum(h, axis=0, keepdims=True)

    @pl.when(i == pl.num_programs(0) - 1)
    def _():
        pooled = acc[...] / N
        c = _swish(jnp.dot(pooled, wq0_ref[...], preferred_element_type=jnp.float32) + bq0_ref[0:1, :])
        coeff = jnp.dot(c, wq1_ref[...], preferred_element_type=jnp.float32) + bq1_ref[0:1, :]
        out_ref[...] = jnp.dot(coeff, x1h_ref[...], preferred_element_type=jnp.float32)


def _final(x2, x1h, pre_p, post_p):
    g = N // NBLK
    cst = lambda shape: pl.BlockSpec(shape, lambda i: tuple(0 for _ in shape))
    return pl.pallas_call(
        _final_body,
        grid=(g,),
        in_specs=[
            pl.BlockSpec((NBLK, D), lambda i: (i, 0)),
            cst((H, H)),
            cst((D, H)), cst((8, H)),
            cst((H, H)), cst((8, H)),
            cst((H, H)), cst((8, H)),
            cst((H, H)), cst((8, H)),
            cst((H, H)), cst((8, H)),
            cst((H, H)), cst((8, H)),
        ],
        out_specs=pl.BlockSpec((8, H), lambda i: (0, 0)),
        out_shape=jax.ShapeDtypeStruct((8, H), jnp.float32),
        scratch_shapes=[pltpu.VMEM((8, H), jnp.float32)],
    )(x2, x1h,
      pre_p['W'][0], _pad8(pre_p['b'][0]),
      pre_p['W'][1], _pad8(pre_p['b'][1]),
      pre_p['W'][2], _pad8(pre_p['b'][2]),
      pre_p['W'][3], _pad8(pre_p['b'][3]),
      post_p['W'][0], _pad8(post_p['b'][0]),
      post_p['W'][1], _pad8(post_p['b'][1]))


def _gather(a, b, dst, src):
    return a[dst], b[src]


def _scatter(m, dst):
    s = jax.ops.segment_sum(m, dst, num_segments=N)
    cnt = jax.ops.segment_sum(jnp.ones((NE,), jnp.float32), dst, num_segments=N)
    s2 = jnp.stack([s, jnp.zeros_like(s)])
    c2 = jnp.stack([jnp.broadcast_to(cnt[:, None], (N, 16)),
                    jnp.zeros((N, 16), jnp.float32)])
    return s2, c2


def kernel(node_feature, vectors, params, edge_index):
    x0 = node_feature[0]
    src = edge_index[0, 0]
    dst = edge_index[0, 1]
    x1 = x0[:, :V]
    x1p = jnp.pad(x1, ((0, 0), (0, H - V)))
    x1h = jnp.pad(x1[:H], ((0, 0), (0, H - V)))

    x2 = x0
    for lp in params['gnn']:
        msg = lp['msg']
        w0 = msg['W'][0]
        w0a = w0[:D]
        w0b = jnp.zeros((D, H), jnp.float32).at[:V].set(w0[D:])
        a, b = _proj(x2, w0a, w0b)
        ga, gb = _gather(a, b, dst, src)
        h1, st1 = _e1(ga, gb, _pad8(msg['b'][0]))
        w1f, b1f = _fold(st1, msg['g'][0], msg['be'][0], msg['W'][1], msg['b'][1])
        h2, st2 = _em(h1, w1f, _pad8(b1f), stats=True)
        w2f, b2f = _fold(st2, msg['g'][1], msg['be'][1], msg['W'][2], msg['b'][2])
        h3, st3 = _em(h2, w2f, _pad8(b2f), stats=True)
        w3f, b3f = _fold(st3, msg['g'][2], msg['be'][2], msg['W'][3], msg['b'][3])
        m = _em(h3, w3f, _pad8(b3f), stats=False)
        s2, c2 = _scatter(m, dst)
        x2 = _upd(x1p, x2, s2, c2, lp['upd'])

    out = _final(x2, x1h, params['pre'], params['post'])
    return out[0, :V]


# speedup vs baseline: 2.1891x; 2.1891x over previous
import functools

import jax
import jax.numpy as jnp
from jax import lax
from jax.experimental import pallas as pl
from jax.experimental.pallas import tpu as pltpu
from jax.experimental.pallas import tpu_sc as plsc

N = 10000
NE = 320000
D = 128
V = 42
H = 64
EBLK = 8000
NBLK = 2000
EPS = 1e-5


def _swish(x):
    return x * jax.nn.sigmoid(x)


def _pad8(b):
    return jnp.zeros((8, H), jnp.float32).at[0].set(b)


def _proj_body(x_ref, wa_ref, wb_ref, a_ref, b_ref):
    x = x_ref[...]
    a_ref[...] = jnp.dot(x, wa_ref[...], preferred_element_type=jnp.float32)
    b_ref[...] = jnp.dot(x, wb_ref[...], preferred_element_type=jnp.float32)


def _proj(x2, w0a, w0b):
    g = N // NBLK
    return pl.pallas_call(
        _proj_body,
        grid=(g,),
        in_specs=[
            pl.BlockSpec((NBLK, D), lambda i: (i, 0)),
            pl.BlockSpec((D, H), lambda i: (0, 0)),
            pl.BlockSpec((D, H), lambda i: (0, 0)),
        ],
        out_specs=[
            pl.BlockSpec((NBLK, H), lambda i: (i, 0)),
            pl.BlockSpec((NBLK, H), lambda i: (i, 0)),
        ],
        out_shape=[
            jax.ShapeDtypeStruct((N, H), jnp.float32),
            jax.ShapeDtypeStruct((N, H), jnp.float32),
        ],
    )(x2, w0a, w0b)


def _e1_body(ga_ref, gb_ref, b_ref, h_ref, st_ref, acc):
    i = pl.program_id(0)
    h = _swish(ga_ref[...] + gb_ref[...] + b_ref[0:1, :])
    h_ref[...] = h

    @pl.when(i == 0)
    def _():
        acc[...] = jnp.zeros_like(acc)

    acc[0:1, :] = acc[0:1, :] + jnp.sum(h, axis=0, keepdims=True)
    acc[1:2, :] = acc[1:2, :] + jnp.sum(h * h, axis=0, keepdims=True)

    @pl.when(i == pl.num_programs(0) - 1)
    def _():
        st_ref[...] = acc[...]


def _e1(ga, gb, b0p):
    g = NE // EBLK
    return pl.pallas_call(
        _e1_body,
        grid=(g,),
        in_specs=[
            pl.BlockSpec((EBLK, H), lambda i: (i, 0)),
            pl.BlockSpec((EBLK, H), lambda i: (i, 0)),
            pl.BlockSpec((8, H), lambda i: (0, 0)),
        ],
        out_specs=[
            pl.BlockSpec((EBLK, H), lambda i: (i, 0)),
            pl.BlockSpec((8, H), lambda i: (0, 0)),
        ],
        out_shape=[
            jax.ShapeDtypeStruct((NE, H), jnp.float32),
            jax.ShapeDtypeStruct((8, H), jnp.float32),
        ],
        scratch_shapes=[pltpu.VMEM((8, H), jnp.float32)],
    )(ga, gb, b0p)


def _em_body(x_ref, w_ref, b_ref, y_ref, st_ref, acc):
    i = pl.program_id(0)
    y = _swish(
        jnp.dot(x_ref[...], w_ref[...], preferred_element_type=jnp.float32)
        + b_ref[0:1, :]
    )
    y_ref[...] = y

    @pl.when(i == 0)
    def _():
        acc[...] = jnp.zeros_like(acc)

    acc[0:1, :] = acc[0:1, :] + jnp.sum(y, axis=0, keepdims=True)
    acc[1:2, :] = acc[1:2, :] + jnp.sum(y * y, axis=0, keepdims=True)

    @pl.when(i == pl.num_programs(0) - 1)
    def _():
        st_ref[...] = acc[...]


def _em_last_body(x_ref, w_ref, b_ref, y_ref):
    y_ref[...] = _swish(
        jnp.dot(x_ref[...], w_ref[...], preferred_element_type=jnp.float32)
        + b_ref[0:1, :]
    )


def _em(x, w, bp, stats):
    g = NE // EBLK
    in_specs = [
        pl.BlockSpec((EBLK, H), lambda i: (i, 0)),
        pl.BlockSpec((H, H), lambda i: (0, 0)),
        pl.BlockSpec((8, H), lambda i: (0, 0)),
    ]
    if stats:
        return pl.pallas_call(
            _em_body,
            grid=(g,),
            in_specs=in_specs,
            out_specs=[
                pl.BlockSpec((EBLK, H), lambda i: (i, 0)),
                pl.BlockSpec((8, H), lambda i: (0, 0)),
            ],
            out_shape=[
                jax.ShapeDtypeStruct((NE, H), jnp.float32),
                jax.ShapeDtypeStruct((8, H), jnp.float32),
            ],
            scratch_shapes=[pltpu.VMEM((8, H), jnp.float32)],
        )(x, w, bp)
    return pl.pallas_call(
        _em_last_body,
        grid=(g,),
        in_specs=in_specs,
        out_specs=pl.BlockSpec((EBLK, H), lambda i: (i, 0)),
        out_shape=jax.ShapeDtypeStruct((NE, H), jnp.float32),
    )(x, w, bp)


def _fold(st, g, be, w_next, b_next):
    mu = st[0] / NE
    var = st[1] / NE - mu * mu
    s = g / jnp.sqrt(var + EPS)
    wf = s[:, None] * w_next
    bf = b_next + (be - mu * s) @ w_next
    return wf, bf


def _upd_body(x1_ref, x2_ref, s2_ref, c2_ref,
              wa_ref, wb_ref, wc_ref, b0_ref, w1_ref, b1_ref,
              w2_ref, b2_ref, w3_ref, b3_ref, xo_ref):
    cnt = c2_ref[0, :, 0:1] + c2_ref[1, :, 0:1]
    cnt = jnp.maximum(cnt, 1.0)
    agg = (s2_ref[0] + s2_ref[1]) / cnt
    t = _swish(
        jnp.dot(x1_ref[...], wa_ref[...], preferred_element_type=jnp.float32)
        + jnp.dot(x2_ref[...], wb_ref[...], preferred_element_type=jnp.float32)
        + jnp.dot(agg, wc_ref[...], preferred_element_type=jnp.float32)
        + b0_ref[0:1, :]
    )
    t = _swish(jnp.dot(t, w1_ref[...], preferred_element_type=jnp.float32) + b1_ref[0:1, :])
    t = _swish(jnp.dot(t, w2_ref[...], preferred_element_type=jnp.float32) + b2_ref[0:1, :])
    t = _swish(jnp.dot(t, w3_ref[...], preferred_element_type=jnp.float32) + b3_ref[0:1, :])
    xo_ref[...] = x2_ref[...] + t


def _upd(x1p, x2, s2, c2, upd_p):
    wu0 = upd_p['W'][0]
    wa = jnp.zeros((H, H), jnp.float32).at[:V].set(wu0[:V])
    wb = wu0[V:V + D]
    wc = wu0[V + D:]
    g = N // NBLK
    cst = lambda shape: pl.BlockSpec(shape, lambda i: tuple(0 for _ in shape))
    return pl.pallas_call(
        _upd_body,
        grid=(g,),
        in_specs=[
            pl.BlockSpec((NBLK, H), lambda i: (i, 0)),
            pl.BlockSpec((NBLK, D), lambda i: (i, 0)),
            pl.BlockSpec((2, NBLK, H), lambda i: (0, i, 0)),
            pl.BlockSpec((2, NBLK, 16), lambda i: (0, i, 0)),
            cst((H, H)), cst((D, H)), cst((H, H)), cst((8, H)),
            cst((H, H)), cst((8, H)),
            cst((H, H)), cst((8, H)),
            cst((H, D)), cst((8, D)),
        ],
        out_specs=pl.BlockSpec((NBLK, D), lambda i: (i, 0)),
        out_shape=jax.ShapeDtypeStruct((N, D), jnp.float32),
    )(x1p, x2, s2, c2,
      wa, wb, wc, _pad8(upd_p['b'][0]),
      upd_p['W'][1], _pad8(upd_p['b'][1]),
      upd_p['W'][2], _pad8(upd_p['b'][2]),
      upd_p['W'][3],
      jnp.zeros((8, D), jnp.float32).at[0].set(upd_p['b'][3]))


def _final_body(x2_ref, x1h_ref,
                wp0_ref, bp0_ref, wp1_ref, bp1_ref, wp2_ref, bp2_ref,
                wp3_ref, bp3_ref, wq0_ref, bq0_ref, wq1_ref, bq1_ref,
                out_ref, acc):
    i = pl.program_id(0)
    t = _swish(jnp.dot(x2_ref[...], wp0_ref[...], preferred_element_type=jnp.float32) + bp0_ref[0:1, :])
    t = _swish(jnp.dot(t, wp1_ref[...], preferred_element_type=jnp.float32) + bp1_ref[0:1, :])
    t = _swish(jnp.dot(t, wp2_ref[...], preferred_element_type=jnp.float32) + bp2_ref[0:1, :])
    h = jnp.dot(t, wp3_ref[...], preferred_element_type=jnp.float32) + bp3_ref[0:1, :]

    @pl.when(i == 0)
    def _():
        acc[...] = jnp.zeros_like(acc)

    acc[0:1, :] = acc[0:1, :] + jnp.sum(h, axis=0, keepdims=True)

    @pl.when(i == pl.num_programs(0) - 1)
    def _():
        pooled = acc[...] / N
        c = _swish(jnp.dot(pooled, wq0_ref[...], preferred_element_type=jnp.float32) + bq0_ref[0:1, :])
        coeff = jnp.dot(c, wq1_ref[...], preferred_element_type=jnp.float32) + bq1_ref[0:1, :]
        out_ref[...] = jnp.dot(coeff, x1h_ref[...], preferred_element_type=jnp.float32)


def _final(x2, x1h, pre_p, post_p):
    g = N // NBLK
    cst = lambda shape: pl.BlockSpec(shape, lambda i: tuple(0 for _ in shape))
    return pl.pallas_call(
        _final_body,
        grid=(g,),
        in_specs=[
            pl.BlockSpec((NBLK, D), lambda i: (i, 0)),
            cst((H, H)),
            cst((D, H)), cst((8, H)),
            cst((H, H)), cst((8, H)),
            cst((H, H)), cst((8, H)),
            cst((H, H)), cst((8, H)),
            cst((H, H)), cst((8, H)),
            cst((H, H)), cst((8, H)),
        ],
        out_specs=pl.BlockSpec((8, H), lambda i: (0, 0)),
        out_shape=jax.ShapeDtypeStruct((8, H), jnp.float32),
        scratch_shapes=[pltpu.VMEM((8, H), jnp.float32)],
    )(x2, x1h,
      pre_p['W'][0], _pad8(pre_p['b'][0]),
      pre_p['W'][1], _pad8(pre_p['b'][1]),
      pre_p['W'][2], _pad8(pre_p['b'][2]),
      pre_p['W'][3], _pad8(pre_p['b'][3]),
      post_p['W'][0], _pad8(post_p['b'][0]),
      post_p['W'][1], _pad8(post_p['b'][1]))


NC = 2
NS = 16
NW = NC * NS
EW = NE // NW
GC = 400


_SC_PARAMS = pltpu.CompilerParams(use_tc_tiling_on_sc=False)


def _gather(a, b, dst, src):
    mesh = plsc.VectorSubcoreMesh(core_axis_name="c", subcore_axis_name="s")

    @functools.partial(
        pl.kernel,
        mesh=mesh,
        out_type=[
            jax.ShapeDtypeStruct((NE, H), jnp.float32),
            jax.ShapeDtypeStruct((NE, H), jnp.float32),
        ],
        scratch_types=[
            pltpu.VMEM((GC,), jnp.int32),
            pltpu.VMEM((GC,), jnp.int32),
            pltpu.VMEM((GC, H), jnp.float32),
            pltpu.VMEM((GC, H), jnp.float32),
            pltpu.SemaphoreType.DMA,
            pltpu.SemaphoreType.DMA,
        ],
        compiler_params=_SC_PARAMS,
    )
    def k(a_hbm, b_hbm, dst_hbm, src_hbm, ga_hbm, gb_hbm, idxd, idxs,
          rowd, rows, sema, semb):
        wid = lax.axis_index("s") * NC + lax.axis_index("c")
        base = wid * EW

        def body(j, carry):
            e0 = base + j * GC
            pltpu.sync_copy(dst_hbm.at[pl.ds(e0, GC)], idxd)
            pltpu.sync_copy(src_hbm.at[pl.ds(e0, GC)], idxs)
            cpa = pltpu.async_copy(a_hbm.at[idxd], rowd, sema)
            cpb = pltpu.async_copy(b_hbm.at[idxs], rows, semb)
            cpa.wait()
            cpb.wait()
            pltpu.sync_copy(rowd, ga_hbm.at[pl.ds(e0, GC)])
            pltpu.sync_copy(rows, gb_hbm.at[pl.ds(e0, GC)])
            return carry

        lax.fori_loop(0, EW // GC, body, 0)

    return k(a, b, dst, src)


def _scatter(m, dst):
    mesh = plsc.VectorSubcoreMesh(core_axis_name="c", subcore_axis_name="s")
    z64 = jnp.zeros((N, H), jnp.float32)
    z16 = jnp.zeros((N, 16), jnp.float32)
    ones = jnp.ones((GC, 16), jnp.float32)
    nrows = N // NS

    @functools.partial(
        pl.kernel,
        mesh=mesh,
        out_type=[
            jax.ShapeDtypeStruct((NC, N, H), jnp.float32),
            jax.ShapeDtypeStruct((NC, N, 16), jnp.float32),
        ],
        scratch_types=[
            pltpu.VMEM((GC,), jnp.int32),
            pltpu.VMEM((GC, H), jnp.float32),
            pltpu.VMEM((GC, 16), jnp.float32),
            pltpu.VMEM_SHARED((N, H), jnp.float32),
            pltpu.VMEM_SHARED((N, 16), jnp.float32),
        ],
        compiler_params=_SC_PARAMS,
    )
    def k(m_hbm, dst_hbm, z64_hbm, z16_hbm, ones_hbm, s_hbm, c_hbm,
          idx, rows, onev, acc, accc):
        cid = lax.axis_index("c")
        sid = lax.axis_index("s")
        wid = sid * NC + cid
        base = wid * EW
        pltpu.sync_copy(ones_hbm, onev)

        @pl.when(sid == 0)
        def _():
            pltpu.sync_copy(z64_hbm, acc)
            pltpu.sync_copy(z16_hbm, accc)

        plsc.subcore_barrier()

        def body(j, carry):
            e0 = base + j * GC
            pltpu.sync_copy(dst_hbm.at[pl.ds(e0, GC)], idx)
            pltpu.sync_copy(m_hbm.at[pl.ds(e0, GC)], rows)
            pltpu.sync_copy(rows, acc.at[idx], add=True)
            pltpu.sync_copy(onev, accc.at[idx], add=True)
            return carry

        lax.fori_loop(0, EW // GC, body, 0)
        plsc.subcore_barrier()
        r0 = sid * nrows
        pltpu.sync_copy(acc.at[pl.ds(r0, nrows)], s_hbm.at[cid, pl.ds(r0, nrows)])
        pltpu.sync_copy(accc.at[pl.ds(r0, nrows)], c_hbm.at[cid, pl.ds(r0, nrows)])

    return k(m, dst, z64, z16, ones)


def kernel(node_feature, vectors, params, edge_index):
    x0 = node_feature[0]
    src = edge_index[0, 0]
    dst = edge_index[0, 1]
    x1 = x0[:, :V]
    x1p = jnp.pad(x1, ((0, 0), (0, H - V)))
    x1h = jnp.pad(x1[:H], ((0, 0), (0, H - V)))

    x2 = x0
    for lp in params['gnn']:
        msg = lp['msg']
        w0 = msg['W'][0]
        w0a = w0[:D]
        w0b = jnp.zeros((D, H), jnp.float32).at[:V].set(w0[D:])
        a, b = _proj(x2, w0a, w0b)
        ga, gb = _gather(a, b, dst, src)
        h1, st1 = _e1(ga, gb, _pad8(msg['b'][0]))
        w1f, b1f = _fold(st1, msg['g'][0], msg['be'][0], msg['W'][1], msg['b'][1])
        h2, st2 = _em(h1, w1f, _pad8(b1f), stats=True)
        w2f, b2f = _fold(st2, msg['g'][1], msg['be'][1], msg['W'][2], msg['b'][2])
        h3, st3 = _em(h2, w2f, _pad8(b2f), stats=True)
        w3f, b3f = _fold(st3, msg['g'][2], msg['be'][2], msg['W'][3], msg['b'][3])
        m = _em(h3, w3f, _pad8(b3f), stats=False)
        s2, c2 = _scatter(m, dst)
        x2 = _upd(x1p, x2, s2, c2, lp['upd'])

    out = _final(x2, x1h, params['pre'], params['post'])
    return out[0, :V]


# speedup vs baseline: 2.2594x; 1.0321x over previous
import functools

import jax
import jax.numpy as jnp
from jax import lax
from jax.experimental import pallas as pl
from jax.experimental.pallas import tpu as pltpu
from jax.experimental.pallas import tpu_sc as plsc

N = 10000
NE = 320000
D = 128
V = 42
H = 64
EBLK = 16000
NBLK = 2000
EPS = 1e-5


def _swish(x):
    return x * (0.5 * jnp.tanh(0.5 * x) + 0.5)


def _pad8(b):
    return jnp.zeros((8, H), jnp.float32).at[0].set(b)


def _proj_body(x_ref, wa_ref, wb_ref, a_ref, b_ref):
    x = x_ref[...]
    a_ref[...] = jnp.dot(x, wa_ref[...], preferred_element_type=jnp.float32)
    b_ref[...] = jnp.dot(x, wb_ref[...], preferred_element_type=jnp.float32)


def _proj(x2, w0a, w0b):
    g = N // NBLK
    return pl.pallas_call(
        _proj_body,
        grid=(g,),
        in_specs=[
            pl.BlockSpec((NBLK, D), lambda i: (i, 0)),
            pl.BlockSpec((D, H), lambda i: (0, 0)),
            pl.BlockSpec((D, H), lambda i: (0, 0)),
        ],
        out_specs=[
            pl.BlockSpec((NBLK, H), lambda i: (i, 0)),
            pl.BlockSpec((NBLK, H), lambda i: (i, 0)),
        ],
        out_shape=[
            jax.ShapeDtypeStruct((N, H), jnp.float32),
            jax.ShapeDtypeStruct((N, H), jnp.float32),
        ],
    )(x2, w0a, w0b)


def _e1_body(ga_ref, gb_ref, b_ref, h_ref, st_ref, acc):
    i = pl.program_id(0)
    h = _swish(ga_ref[...] + gb_ref[...] + b_ref[0:1, :])
    h_ref[...] = h

    @pl.when(i == 0)
    def _():
        acc[...] = jnp.zeros_like(acc)

    acc[0:1, :] = acc[0:1, :] + jnp.sum(h, axis=0, keepdims=True)
    acc[1:2, :] = acc[1:2, :] + jnp.sum(h * h, axis=0, keepdims=True)

    @pl.when(i == pl.num_programs(0) - 1)
    def _():
        st_ref[...] = acc[...]


def _e1(ga, gb, b0p):
    g = NE // EBLK
    return pl.pallas_call(
        _e1_body,
        grid=(g,),
        in_specs=[
            pl.BlockSpec((EBLK, H), lambda i: (i, 0)),
            pl.BlockSpec((EBLK, H), lambda i: (i, 0)),
            pl.BlockSpec((8, H), lambda i: (0, 0)),
        ],
        out_specs=[
            pl.BlockSpec((EBLK, H), lambda i: (i, 0)),
            pl.BlockSpec((8, H), lambda i: (0, 0)),
        ],
        out_shape=[
            jax.ShapeDtypeStruct((NE, H), jnp.float32),
            jax.ShapeDtypeStruct((8, H), jnp.float32),
        ],
        scratch_shapes=[pltpu.VMEM((8, H), jnp.float32)],
    )(ga, gb, b0p)


def _em_body(x_ref, w_ref, b_ref, y_ref, st_ref, acc):
    i = pl.program_id(0)
    y = _swish(
        jnp.dot(x_ref[...], w_ref[...], preferred_element_type=jnp.float32)
        + b_ref[0:1, :]
    )
    y_ref[...] = y

    @pl.when(i == 0)
    def _():
        acc[...] = jnp.zeros_like(acc)

    acc[0:1, :] = acc[0:1, :] + jnp.sum(y, axis=0, keepdims=True)
    acc[1:2, :] = acc[1:2, :] + jnp.sum(y * y, axis=0, keepdims=True)

    @pl.when(i == pl.num_programs(0) - 1)
    def _():
        st_ref[...] = acc[...]


def _em_last_body(x_ref, w_ref, b_ref, y_ref):
    y_ref[...] = _swish(
        jnp.dot(x_ref[...], w_ref[...], preferred_element_type=jnp.float32)
        + b_ref[0:1, :]
    )


def _em(x, w, bp, stats):
    g = NE // EBLK
    in_specs = [
        pl.BlockSpec((EBLK, H), lambda i: (i, 0)),
        pl.BlockSpec((H, H), lambda i: (0, 0)),
        pl.BlockSpec((8, H), lambda i: (0, 0)),
    ]
    if stats:
        return pl.pallas_call(
            _em_body,
            grid=(g,),
            in_specs=in_specs,
            out_specs=[
                pl.BlockSpec((EBLK, H), lambda i: (i, 0)),
                pl.BlockSpec((8, H), lambda i: (0, 0)),
            ],
            out_shape=[
                jax.ShapeDtypeStruct((NE, H), jnp.float32),
                jax.ShapeDtypeStruct((8, H), jnp.float32),
            ],
            scratch_shapes=[pltpu.VMEM((8, H), jnp.float32)],
        )(x, w, bp)
    return pl.pallas_call(
        _em_last_body,
        grid=(g,),
        in_specs=in_specs,
        out_specs=pl.BlockSpec((EBLK, H), lambda i: (i, 0)),
        out_shape=jax.ShapeDtypeStruct((NE, H), jnp.float32),
    )(x, w, bp)


def _fold(st, g, be, w_next, b_next):
    mu = st[0] / NE
    var = st[1] / NE - mu * mu
    s = g / jnp.sqrt(var + EPS)
    wf = s[:, None] * w_next
    bf = b_next + (be - mu * s) @ w_next
    return wf, bf


def _upd_body(x1_ref, x2_ref, s2_ref, c2_ref,
              wa_ref, wb_ref, wc_ref, b0_ref, w1_ref, b1_ref,
              w2_ref, b2_ref, w3_ref, b3_ref, xo_ref):
    cnt = c2_ref[0, :, 0:1] + c2_ref[1, :, 0:1]
    cnt = jnp.maximum(cnt, 1.0)
    agg = (s2_ref[0] + s2_ref[1]) / cnt
    t = _swish(
        jnp.dot(x1_ref[...], wa_ref[...], preferred_element_type=jnp.float32)
        + jnp.dot(x2_ref[...], wb_ref[...], preferred_element_type=jnp.float32)
        + jnp.dot(agg, wc_ref[...], preferred_element_type=jnp.float32)
        + b0_ref[0:1, :]
    )
    t = _swish(jnp.dot(t, w1_ref[...], preferred_element_type=jnp.float32) + b1_ref[0:1, :])
    t = _swish(jnp.dot(t, w2_ref[...], preferred_element_type=jnp.float32) + b2_ref[0:1, :])
    t = _swish(jnp.dot(t, w3_ref[...], preferred_element_type=jnp.float32) + b3_ref[0:1, :])
    xo_ref[...] = x2_ref[...] + t


def _upd(x1p, x2, s2, c2, upd_p):
    wu0 = upd_p['W'][0]
    wa = jnp.zeros((H, H), jnp.float32).at[:V].set(wu0[:V])
    wb = wu0[V:V + D]
    wc = wu0[V + D:]
    g = N // NBLK
    cst = lambda shape: pl.BlockSpec(shape, lambda i: tuple(0 for _ in shape))
    return pl.pallas_call(
        _upd_body,
        grid=(g,),
        in_specs=[
            pl.BlockSpec((NBLK, H), lambda i: (i, 0)),
            pl.BlockSpec((NBLK, D), lambda i: (i, 0)),
            pl.BlockSpec((2, NBLK, H), lambda i: (0, i, 0)),
            pl.BlockSpec((2, NBLK, 16), lambda i: (0, i, 0)),
            cst((H, H)), cst((D, H)), cst((H, H)), cst((8, H)),
            cst((H, H)), cst((8, H)),
            cst((H, H)), cst((8, H)),
            cst((H, D)), cst((8, D)),
        ],
        out_specs=pl.BlockSpec((NBLK, D), lambda i: (i, 0)),
        out_shape=jax.ShapeDtypeStruct((N, D), jnp.float32),
    )(x1p, x2, s2, c2,
      wa, wb, wc, _pad8(upd_p['b'][0]),
      upd_p['W'][1], _pad8(upd_p['b'][1]),
      upd_p['W'][2], _pad8(upd_p['b'][2]),
      upd_p['W'][3],
      jnp.zeros((8, D), jnp.float32).at[0].set(upd_p['b'][3]))


def _final_body(x2_ref, x1h_ref,
                wp0_ref, bp0_ref, wp1_ref, bp1_ref, wp2_ref, bp2_ref,
                wp3_ref, bp3_ref, wq0_ref, bq0_ref, wq1_ref, bq1_ref,
                out_ref, acc):
    i = pl.program_id(0)
    t = _swish(jnp.dot(x2_ref[...], wp0_ref[...], preferred_element_type=jnp.float32) + bp0_ref[0:1, :])
    t = _swish(jnp.dot(t, wp1_ref[...], preferred_element_type=jnp.float32) + bp1_ref[0:1, :])
    t = _swish(jnp.dot(t, wp2_ref[...], preferred_element_type=jnp.float32) + bp2_ref[0:1, :])
    h = jnp.dot(t, wp3_ref[...], preferred_element_type=jnp.float32) + bp3_ref[0:1, :]

    @pl.when(i == 0)
    def _():
        acc[...] = jnp.zeros_like(acc)

    acc[0:1, :] = acc[0:1, :] + jnp.sum(h, axis=0, keepdims=True)

    @pl.when(i == pl.num_programs(0) - 1)
    def _():
        pooled = acc[...] / N
        c = _swish(jnp.dot(pooled, wq0_ref[...], preferred_element_type=jnp.float32) + bq0_ref[0:1, :])
        coeff = jnp.dot(c, wq1_ref[...], preferred_element_type=jnp.float32) + bq1_ref[0:1, :]
        out_ref[...] = jnp.dot(coeff, x1h_ref[...], preferred_element_type=jnp.float32)


def _final(x2, x1h, pre_p, post_p):
    g = N // NBLK
    cst = lambda shape: pl.BlockSpec(shape, lambda i: tuple(0 for _ in shape))
    return pl.pallas_call(
        _final_body,
        grid=(g,),
        in_specs=[
            pl.BlockSpec((NBLK, D), lambda i: (i, 0)),
            cst((H, H)),
            cst((D, H)), cst((8, H)),
            cst((H, H)), cst((8, H)),
            cst((H, H)), cst((8, H)),
            cst((H, H)), cst((8, H)),
            cst((H, H)), cst((8, H)),
            cst((H, H)), cst((8, H)),
        ],
        out_specs=pl.BlockSpec((8, H), lambda i: (0, 0)),
        out_shape=jax.ShapeDtypeStruct((8, H), jnp.float32),
        scratch_shapes=[pltpu.VMEM((8, H), jnp.float32)],
    )(x2, x1h,
      pre_p['W'][0], _pad8(pre_p['b'][0]),
      pre_p['W'][1], _pad8(pre_p['b'][1]),
      pre_p['W'][2], _pad8(pre_p['b'][2]),
      pre_p['W'][3], _pad8(pre_p['b'][3]),
      post_p['W'][0], _pad8(post_p['b'][0]),
      post_p['W'][1], _pad8(post_p['b'][1]))


NC = 2
NS = 16
NW = NC * NS
EW = NE // NW
GC = 400


_SC_PARAMS = pltpu.CompilerParams(use_tc_tiling_on_sc=False)


def _gather(a, b, dst, src):
    mesh = plsc.VectorSubcoreMesh(core_axis_name="c", subcore_axis_name="s")

    @functools.partial(
        pl.kernel,
        mesh=mesh,
        out_type=[
            jax.ShapeDtypeStruct((NE, H), jnp.float32),
            jax.ShapeDtypeStruct((NE, H), jnp.float32),
        ],
        scratch_types=[
            pltpu.VMEM((GC,), jnp.int32),
            pltpu.VMEM((GC,), jnp.int32),
            pltpu.VMEM((GC, H), jnp.float32),
            pltpu.VMEM((GC, H), jnp.float32),
            pltpu.SemaphoreType.DMA,
            pltpu.SemaphoreType.DMA,
        ],
        compiler_params=_SC_PARAMS,
    )
    def k(a_hbm, b_hbm, dst_hbm, src_hbm, ga_hbm, gb_hbm, idxd, idxs,
          rowd, rows, sema, semb):
        wid = lax.axis_index("s") * NC + lax.axis_index("c")
        base = wid * EW

        def body(j, carry):
            e0 = base + j * GC
            pltpu.sync_copy(dst_hbm.at[pl.ds(e0, GC)], idxd)
            pltpu.sync_copy(src_hbm.at[pl.ds(e0, GC)], idxs)
            cpa = pltpu.async_copy(a_hbm.at[idxd], rowd, sema)
            cpb = pltpu.async_copy(b_hbm.at[idxs], rows, semb)
            cpa.wait()
            cpb.wait()
            pltpu.sync_copy(rowd, ga_hbm.at[pl.ds(e0, GC)])
            pltpu.sync_copy(rows, gb_hbm.at[pl.ds(e0, GC)])
            return carry

        lax.fori_loop(0, EW // GC, body, 0)

    return k(a, b, dst, src)


def _scatter(m, dst):
    mesh = plsc.VectorSubcoreMesh(core_axis_name="c", subcore_axis_name="s")
    z64 = jnp.zeros((N, H), jnp.float32)
    z16 = jnp.zeros((N, 16), jnp.float32)
    ones = jnp.ones((GC, 16), jnp.float32)
    nrows = N // NS

    @functools.partial(
        pl.kernel,
        mesh=mesh,
        out_type=[
            jax.ShapeDtypeStruct((NC, N, H), jnp.float32),
            jax.ShapeDtypeStruct((NC, N, 16), jnp.float32),
        ],
        scratch_types=[
            pltpu.VMEM((GC,), jnp.int32),
            pltpu.VMEM((GC, H), jnp.float32),
            pltpu.VMEM((GC, 16), jnp.float32),
            pltpu.VMEM_SHARED((N, H), jnp.float32),
            pltpu.VMEM_SHARED((N, 16), jnp.float32),
        ],
        compiler_params=_SC_PARAMS,
    )
    def k(m_hbm, dst_hbm, z64_hbm, z16_hbm, ones_hbm, s_hbm, c_hbm,
          idx, rows, onev, acc, accc):
        cid = lax.axis_index("c")
        sid = lax.axis_index("s")
        wid = sid * NC + cid
        base = wid * EW
        pltpu.sync_copy(ones_hbm, onev)

        @pl.when(sid == 0)
        def _():
            pltpu.sync_copy(z64_hbm, acc)
            pltpu.sync_copy(z16_hbm, accc)

        plsc.subcore_barrier()

        def body(j, carry):
            e0 = base + j * GC
            pltpu.sync_copy(dst_hbm.at[pl.ds(e0, GC)], idx)
            pltpu.sync_copy(m_hbm.at[pl.ds(e0, GC)], rows)
            pltpu.sync_copy(rows, acc.at[idx], add=True)
            pltpu.sync_copy(onev, accc.at[idx], add=True)
            return carry

        lax.fori_loop(0, EW // GC, body, 0)
        plsc.subcore_barrier()
        r0 = sid * nrows
        pltpu.sync_copy(acc.at[pl.ds(r0, nrows)], s_hbm.at[cid, pl.ds(r0, nrows)])
        pltpu.sync_copy(accc.at[pl.ds(r0, nrows)], c_hbm.at[cid, pl.ds(r0, nrows)])

    return k(m, dst, z64, z16, ones)


def kernel(node_feature, vectors, params, edge_index):
    x0 = node_feature[0]
    src = edge_index[0, 0]
    dst = edge_index[0, 1]
    x1 = x0[:, :V]
    x1p = jnp.pad(x1, ((0, 0), (0, H - V)))
    x1h = jnp.pad(x1[:H], ((0, 0), (0, H - V)))

    x2 = x0
    for lp in params['gnn']:
        msg = lp['msg']
        w0 = msg['W'][0]
        w0a = w0[:D]
        w0b = jnp.zeros((D, H), jnp.float32).at[:V].set(w0[D:])
        a, b = _proj(x2, w0a, w0b)
        ga, gb = _gather(a, b, dst, src)
        h1, st1 = _e1(ga, gb, _pad8(msg['b'][0]))
        w1f, b1f = _fold(st1, msg['g'][0], msg['be'][0], msg['W'][1], msg['b'][1])
        h2, st2 = _em(h1, w1f, _pad8(b1f), stats=True)
        w2f, b2f = _fold(st2, msg['g'][1], msg['be'][1], msg['W'][2], msg['b'][2])
        h3, st3 = _em(h2, w2f, _pad8(b2f), stats=True)
        w3f, b3f = _fold(st3, msg['g'][2], msg['be'][2], msg['W'][3], msg['b'][3])
        m = _em(h3, w3f, _pad8(b3f), stats=False)
        s2, c2 = _scatter(m, dst)
        x2 = _upd(x1p, x2, s2, c2, lp['upd'])

    out = _final(x2, x1h, params['pre'], params['post'])
    return out[0, :V]
